# Initial kernel scaffold; baseline (speedup 1.0000x reference)
import jax
import jax.numpy as jnp
from jax.experimental import pallas as pl


def kernel(users, items, edge_row, edge_col, edge_val, user_age, user_gender, user_occ, user_zip, item_title, item_temp, item_hum, item_wind, user_id_tab, user_age_tab, user_gender_tab, user_occ_tab, user_zip_tab, item_id_tab, item_title_tab, item_temp_tab, item_hum_tab, item_wind_tab, W1, b1, W2, b2, Wt, bt):
    raise NotImplementedError("write your pallas kernel here")


# speedup vs baseline: 4.8407x; 4.8407x over previous
import functools

import jax
import jax.numpy as jnp
import numpy as np
from jax import lax
from jax.experimental import pallas as pl
from jax.experimental.pallas import tpu as pltpu
from jax.experimental.pallas import tpu_sc as plsc

NU = 25000
NI = 25000
NN = NU + NI
EH = 400000
DD = 64
BB = 4096

NCORE = 2
NSUB = 16
LANE = 16

def _bcast_lane(v16, e):
  return lax.broadcast_in_dim(v16[e], (16,), ())


_U_AB, _U_GB, _U_OB, _U_ZB = 0, 480, 484, 684
_I_TB, _I_TEB, _I_HB, _I_WB = 0, 16000, 16400, 17200
_TU_LEN = 18688
_TI_LEN = 17520

_EW = 40
_EBUF = 48

def _user_consts():
  i16 = lax.iota(jnp.int32, 16)
  in_a = i16 < 4
  in_g = jnp.logical_and(i16 >= 4, i16 < 6)
  in_o = jnp.logical_and(i16 >= 6, i16 < 14)
  c1 = jnp.where(in_a, i16,
                 jnp.where(in_g, i16 + (_U_GB - 4),
                           jnp.where(in_o, i16 + (_U_OB - 6),
                                     i16 + (_U_ZB - 14))))
  m_a = jnp.where(in_a, 4, 0)
  m_g = jnp.where(in_g, 2, 0)
  m_o = jnp.where(in_o, 8, 0)
  m_z1 = jnp.where(i16 >= 14, 18, 0)
  c2 = i16 + (_U_ZB + 2)
  return c1, m_a, m_g, m_o, m_z1, c2


def _item_consts():
  i16 = lax.iota(jnp.int32, 16)
  lo = i16 < 8
  c1 = jnp.where(lo, i16, i16 + (_I_TEB - 8))
  m_t = jnp.where(lo, 8, 0)
  m_te = 8 - m_t
  c2 = jnp.where(lo, i16 + _I_HB, i16 + (_I_WB - 8))
  return c1, m_t, m_te, c2


def _embed_body(age_h, gen_h, occ_h, zip_h, tit_h, tem_h, hum_h, win_h,
                tu_h, ti_h, out_h,
                tab_v, i0_v, i1_v, i2_v, i3_v, out_v):
  c = lax.axis_index("c")
  s = lax.axis_index("s")
  zero16 = lax.iota(jnp.int32, 16) * 0
  for buf in (i0_v, i1_v, i2_v, i3_v):
    buf[pl.ds(32, 16)] = zero16

  @pl.when(c == 0)
  def _users():
    u_c1, u_ma, u_mg, u_mo, u_mz1, u_c2 = _user_consts()
    pltpu.sync_copy(tu_h, tab_v.at[pl.ds(0, _TU_LEN)])

    @pl.loop(s, 625, step=NSUB)
    def _win(w):
      base = w * _EW
      pltpu.sync_copy(age_h.at[pl.ds(base, _EW)], i0_v.at[pl.ds(0, _EW)])
      pltpu.sync_copy(gen_h.at[pl.ds(base, _EW)], i1_v.at[pl.ds(0, _EW)])
      pltpu.sync_copy(occ_h.at[pl.ds(base, _EW)], i2_v.at[pl.ds(0, _EW)])
      pltpu.sync_copy(zip_h.at[pl.ds(base, _EW)], i3_v.at[pl.ds(0, _EW)])
      for g in range(_EBUF // 16):
        a16 = i0_v[pl.ds(g * 16, 16)]
        g16 = i1_v[pl.ds(g * 16, 16)]
        o16 = i2_v[pl.ds(g * 16, 16)]
        z16 = i3_v[pl.ds(g * 16, 16)]
        for e in range(16):
          u = g * 16 + e
          if u >= _EW:
            break
          ab = _bcast_lane(a16, e)
          gb = _bcast_lane(g16, e)
          ob = _bcast_lane(o16, e)
          zb = _bcast_lane(z16, e)
          addr1 = u_c1 + ab * u_ma + gb * u_mg + ob * u_mo + zb * u_mz1
          addr2 = u_c2 + zb * 18
          out_v[u, pl.ds(0, 16)] = plsc.load_gather(tab_v, [addr1])
          out_v[u, pl.ds(16, 16)] = plsc.load_gather(tab_v, [addr2])
      pltpu.sync_copy(out_v.at[pl.ds(0, _EW)], out_h.at[pl.ds(base, _EW)])

  @pl.when(c == 1)
  def _items():
    i_c1, i_mt, i_mte, i_c2 = _item_consts()
    pltpu.sync_copy(ti_h, tab_v.at[pl.ds(0, _TI_LEN)])

    @pl.loop(s, 625, step=NSUB)
    def _win(w):
      base = w * _EW
      pltpu.sync_copy(tit_h.at[pl.ds(base, _EW)], i0_v.at[pl.ds(0, _EW)])
      pltpu.sync_copy(tem_h.at[pl.ds(base, _EW)], i1_v.at[pl.ds(0, _EW)])
      pltpu.sync_copy(hum_h.at[pl.ds(base, _EW)], i2_v.at[pl.ds(0, _EW)])
      pltpu.sync_copy(win_h.at[pl.ds(base, _EW)], i3_v.at[pl.ds(0, _EW)])
      for g in range(_EBUF // 16):
        t16 = i0_v[pl.ds(g * 16, 16)]
        te16 = i1_v[pl.ds(g * 16, 16)]
        h16 = i2_v[pl.ds(g * 16, 16)]
        w16 = i3_v[pl.ds(g * 16, 16)]
        for e in range(16):
          u = g * 16 + e
          if u >= _EW:
            break
          tb = _bcast_lane(t16, e)
          teb = _bcast_lane(te16, e)
          hb = _bcast_lane(h16, e)
          wb = _bcast_lane(w16, e)
          addr1 = i_c1 + tb * i_mt + teb * i_mte
          addr2 = i_c2 + hb * i_mt + wb * i_mte
          out_v[u, pl.ds(0, 16)] = plsc.load_gather(tab_v, [addr1])
          out_v[u, pl.ds(16, 16)] = plsc.load_gather(tab_v, [addr2])
      pltpu.sync_copy(out_v.at[pl.ds(0, _EW)],
                      out_h.at[pl.ds(NU + base, _EW)])


def _embed_call(age, gen, occ, zp, tit, tem, hum, win, tu, ti):
  mesh = plsc.VectorSubcoreMesh(core_axis_name="c", subcore_axis_name="s")
  f = pl.kernel(
      _embed_body,
      out_type=jax.ShapeDtypeStruct((NN, 32), jnp.float32),
      mesh=mesh,
      compiler_params=pltpu.CompilerParams(needs_layout_passes=False, use_tc_tiling_on_sc=False),
      scratch_types=[
          pltpu.MemorySpace.VMEM((_TU_LEN,), jnp.float32),
          pltpu.MemorySpace.VMEM((_EBUF,), jnp.int32),
          pltpu.MemorySpace.VMEM((_EBUF,), jnp.int32),
          pltpu.MemorySpace.VMEM((_EBUF,), jnp.int32),
          pltpu.MemorySpace.VMEM((_EBUF,), jnp.int32),
          pltpu.MemorySpace.VMEM((_EBUF, 32), jnp.float32),
      ],
  )
  return f(age, gen, occ, zp, tit, tem, hum, win, tu, ti)


_W = 128
_NWIN = EH // _W


def _spmm_body(ecol_h, eval_h, erow_h, feat_h, out_h,
               col_v, val_v, row_v, loc_v, rows_v, zz_v, g_sh, sem):
  c = lax.axis_index("c")
  s = lax.axis_index("s")

  zf = (lax.iota(jnp.int32, 16) * 0).astype(jnp.float32)
  for i in range(25):
    for j in range(4):
      zz_v[i, pl.ds(j * 16, 16)] = zf

  @pl.loop(s, 1000, step=NSUB)
  def _zero(i):
    pltpu.sync_copy(zz_v, g_sh.at[pl.ds(i * 25, 25)])

  plsc.subcore_barrier()

  cnu = c * NU

  @pl.loop(s, _NWIN, step=NSUB)
  def _win(w):
    base = c * EH + w * _W
    pltpu.sync_copy(ecol_h.at[pl.ds(base, _W)], col_v)
    pltpu.sync_copy(eval_h.at[pl.ds(base, _W)], val_v)
    pltpu.sync_copy(erow_h.at[pl.ds(base, _W)], row_v)
    for g in range(_W // 16):
      loc_v[pl.ds(g * 16, 16)] = row_v[pl.ds(g * 16, 16)] - cnu
    pltpu.async_copy(feat_h.at[col_v], rows_v, sem).wait()
    for g in range(_W // 16):
      v16 = val_v[pl.ds(g * 16, 16)]
      for e in range(16):
        r = g * 16 + e
        vb = _bcast_lane(v16, e)
        for j in range(4):
          rows_v[r, pl.ds(j * 16, 16)] = rows_v[r, pl.ds(j * 16, 16)] * vb
    pltpu.sync_copy(rows_v, g_sh.at[loc_v], add=True)

  plsc.subcore_barrier()

  @pl.loop(s, 625, step=NSUB)
  def _flush(i):
    pltpu.sync_copy(g_sh.at[pl.ds(i * 40, 40)],
                    out_h.at[pl.ds(cnu + i * 40, 40)])


def _spmm_call(ecol, evl, erow, feat):
  mesh = plsc.VectorSubcoreMesh(core_axis_name="c", subcore_axis_name="s")
  f = pl.kernel(
      _spmm_body,
      out_type=jax.ShapeDtypeStruct((NN, DD), jnp.float32),
      mesh=mesh,
      compiler_params=pltpu.CompilerParams(needs_layout_passes=False, use_tc_tiling_on_sc=False),
      scratch_types=[
          pltpu.MemorySpace.VMEM((_W,), jnp.int32),
          pltpu.MemorySpace.VMEM((_W,), jnp.float32),
          pltpu.MemorySpace.VMEM((_W,), jnp.int32),
          pltpu.MemorySpace.VMEM((_W,), jnp.int32),
          pltpu.MemorySpace.VMEM((_W, DD), jnp.float32),
          pltpu.MemorySpace.VMEM((25, DD), jnp.float32),
          pltpu.MemorySpace.VMEM_SHARED((NU, DD), jnp.float32),
          pltpu.SemaphoreType.DMA,
      ],
  )
  return f(ecol, evl, erow, feat)


_GW = BB // (NCORE * NSUB)


def _gather_body(users_h, items_h, f0_h, f1_h, f2_h,
                 ou0_h, ou1_h, ou2_h, oi0_h, oi1_h, oi2_h,
                 idx_v, rows_v, sem):
  c = lax.axis_index("c")
  s = lax.axis_index("s")
  w = s * NCORE + c
  base = w * _GW

  pltpu.sync_copy(users_h.at[pl.ds(base, _GW)], idx_v)
  for (f_h, o_h) in ((f0_h, ou0_h), (f1_h, ou1_h), (f2_h, ou2_h)):
    pltpu.async_copy(f_h.at[idx_v], rows_v, sem).wait()
    pltpu.sync_copy(rows_v, o_h.at[pl.ds(base, _GW)])

  pltpu.sync_copy(items_h.at[pl.ds(base, _GW)], idx_v)
  for g in range(_GW // 16):
    idx_v[pl.ds(g * 16, 16)] = idx_v[pl.ds(g * 16, 16)] + NU
  for (f_h, o_h) in ((f0_h, oi0_h), (f1_h, oi1_h), (f2_h, oi2_h)):
    pltpu.async_copy(f_h.at[idx_v], rows_v, sem).wait()
    pltpu.sync_copy(rows_v, o_h.at[pl.ds(base, _GW)])


def _gather_call(users, items, f0, f1, f2):
  mesh = plsc.VectorSubcoreMesh(core_axis_name="c", subcore_axis_name="s")
  ot = jax.ShapeDtypeStruct((BB, DD), jnp.float32)
  f = pl.kernel(
      _gather_body,
      out_type=(ot, ot, ot, ot, ot, ot),
      mesh=mesh,
      compiler_params=pltpu.CompilerParams(needs_layout_passes=False, use_tc_tiling_on_sc=False),
      scratch_types=[
          pltpu.MemorySpace.VMEM((_GW,), jnp.int32),
          pltpu.MemorySpace.VMEM((_GW, DD), jnp.float32),
          pltpu.SemaphoreType.DMA,
      ],
  )
  return f(users, items, f0, f1, f2)


_RB = 1000


def _dense_body(g_ref, f_ref, w1_ref, w2_ref, b1_ref, b2_ref, o_ref):
  g = g_ref[...]
  f = f_ref[...]
  dn = (((1,), (1,)), ((), ()))
  acc = lax.dot_general(g + f, w1_ref[...], dn,
                        preferred_element_type=jnp.float32)
  acc = acc + lax.dot_general(g * f, w2_ref[...], dn,
                              preferred_element_type=jnp.float32)
  acc = acc + b1_ref[...] + b2_ref[...]
  o_ref[...] = jnp.where(acc > 0, acc, 0.01 * acc)


def _dense_call(g, f, w1, w2, b1, b2):
  grid = (NN // _RB,)
  return pl.pallas_call(
      _dense_body,
      grid=grid,
      in_specs=[
          pl.BlockSpec((_RB, DD), lambda i: (i, 0)),
          pl.BlockSpec((_RB, DD), lambda i: (i, 0)),
          pl.BlockSpec((DD, DD), lambda i: (0, 0)),
          pl.BlockSpec((DD, DD), lambda i: (0, 0)),
          pl.BlockSpec((1, DD), lambda i: (0, 0)),
          pl.BlockSpec((1, DD), lambda i: (0, 0)),
      ],
      out_specs=pl.BlockSpec((_RB, DD), lambda i: (i, 0)),
      out_shape=jax.ShapeDtypeStruct((NN, DD), jnp.float32),
  )(g, f, w1, w2, b1.reshape(1, DD), b2.reshape(1, DD))


def _final_body(u0_ref, u1_ref, u2_ref, i0_ref, i1_ref, i2_ref,
                wt_ref, bt_ref, o_ref):
  wt = wt_ref[...]
  dn = (((1,), (1,)), ((), ()))
  ue = bt_ref[...]
  ie = bt_ref[...]
  for l, (u_ref, i_ref) in enumerate(((u0_ref, i0_ref), (u1_ref, i1_ref),
                                      (u2_ref, i2_ref))):
    wtl = wt[:, l * DD:(l + 1) * DD]
    ue = ue + lax.dot_general(u_ref[...], wtl, dn,
                              preferred_element_type=jnp.float32)
    ie = ie + lax.dot_general(i_ref[...], wtl, dn,
                              preferred_element_type=jnp.float32)
  o_ref[...] = jnp.sum(ue * ie, axis=1)


def _final_call(u0, u1, u2, i0, i1, i2, wt, bt):
  return pl.pallas_call(
      _final_body,
      grid=(1,),
      in_specs=[pl.BlockSpec((BB, DD), lambda i: (0, 0))] * 6
      + [pl.BlockSpec((DD, 3 * DD), lambda i: (0, 0)),
         pl.BlockSpec((1, DD), lambda i: (0, 0))],
      out_specs=pl.BlockSpec((BB,), lambda i: (0,)),
      out_shape=jax.ShapeDtypeStruct((BB,), jnp.float32),
  )(u0, u1, u2, i0, i1, i2, wt, bt.reshape(1, DD))


def kernel(users, items, edge_row, edge_col, edge_val, user_age, user_gender,
           user_occ, user_zip, item_title, item_temp, item_hum, item_wind,
           user_id_tab, user_age_tab, user_gender_tab, user_occ_tab,
           user_zip_tab, item_id_tab, item_title_tab, item_temp_tab,
           item_hum_tab, item_wind_tab, W1, b1, W2, b2, Wt, bt):
  tu = jnp.concatenate([
      user_age_tab.reshape(-1), user_gender_tab.reshape(-1),
      user_occ_tab.reshape(-1), user_zip_tab.reshape(-1),
      jnp.zeros((_TU_LEN - 18684,), jnp.float32)])
  ti = jnp.concatenate([
      item_title_tab.reshape(-1), item_temp_tab.reshape(-1),
      item_hum_tab.reshape(-1), item_wind_tab.reshape(-1)])

  emb32 = _embed_call(
      user_age.astype(jnp.int32), user_gender.astype(jnp.int32),
      user_occ.astype(jnp.int32), user_zip.astype(jnp.int32),
      item_title.astype(jnp.int32), item_temp.astype(jnp.int32),
      item_hum.astype(jnp.int32), item_wind.astype(jnp.int32), tu, ti)

  ids = jnp.concatenate([user_id_tab, item_id_tab], axis=0)
  feat = jnp.concatenate([ids, emb32], axis=1)

  ecol = edge_col.astype(jnp.int32)
  erow = edge_row.astype(jnp.int32)

  feats = [feat]
  for l in range(2):
    g = _spmm_call(ecol, edge_val, erow, feat)
    feat = _dense_call(g, feat, W1[l], W2[l], b1[l], b2[l])
    feats.append(feat)

  gu0, gu1, gu2, gi0, gi1, gi2 = _gather_call(
      users.astype(jnp.int32), items.astype(jnp.int32),
      feats[0], feats[1], feats[2])

  return _final_call(gu0, gu1, gu2, gi0, gi1, gi2, Wt, bt)


# speedup vs baseline: 8.9490x; 1.8487x over previous
import functools

import jax
import jax.numpy as jnp
import numpy as np
from jax import lax
from jax.experimental import pallas as pl
from jax.experimental.pallas import tpu as pltpu
from jax.experimental.pallas import tpu_sc as plsc

NU = 25000
NI = 25000
NN = NU + NI
EH = 400000
DD = 64
BB = 4096

NCORE = 2
NSUB = 16
LANE = 16

def _bcast_lane(v16, e):
  return lax.broadcast_in_dim(v16[e], (16,), ())


_U_AB, _U_GB, _U_OB, _U_ZB = 0, 480, 484, 684
_I_TB, _I_TEB, _I_HB, _I_WB = 0, 16000, 16400, 17200
_TU_LEN = 18688
_TI_LEN = 17520

_EW = 40
_EBUF = 48

def _user_consts():
  i16 = lax.iota(jnp.int32, 16)
  in_a = i16 < 4
  in_g = jnp.logical_and(i16 >= 4, i16 < 6)
  in_o = jnp.logical_and(i16 >= 6, i16 < 14)
  c1 = jnp.where(in_a, i16,
                 jnp.where(in_g, i16 + (_U_GB - 4),
                           jnp.where(in_o, i16 + (_U_OB - 6),
                                     i16 + (_U_ZB - 14))))
  m_a = jnp.where(in_a, 4, 0)
  m_g = jnp.where(in_g, 2, 0)
  m_o = jnp.where(in_o, 8, 0)
  m_z1 = jnp.where(i16 >= 14, 18, 0)
  c2 = i16 + (_U_ZB + 2)
  return c1, m_a, m_g, m_o, m_z1, c2


def _item_consts():
  i16 = lax.iota(jnp.int32, 16)
  lo = i16 < 8
  c1 = jnp.where(lo, i16, i16 + (_I_TEB - 8))
  m_t = jnp.where(lo, 8, 0)
  m_te = 8 - m_t
  c2 = jnp.where(lo, i16 + _I_HB, i16 + (_I_WB - 8))
  return c1, m_t, m_te, c2


def _embed_body(age_h, gen_h, occ_h, zip_h, tit_h, tem_h, hum_h, win_h,
                tu_h, ti_h, out_h,
                tab_v, i0_v, i1_v, i2_v, i3_v, out_v):
  c = lax.axis_index("c")
  s = lax.axis_index("s")
  zero16 = lax.iota(jnp.int32, 16) * 0
  for buf in (i0_v, i1_v, i2_v, i3_v):
    buf[pl.ds(32, 16)] = zero16

  @pl.when(c == 0)
  def _users():
    u_c1, u_ma, u_mg, u_mo, u_mz1, u_c2 = _user_consts()
    pltpu.sync_copy(tu_h, tab_v.at[pl.ds(0, _TU_LEN)])

    @pl.loop(s, 625, step=NSUB)
    def _win(w):
      base = w * _EW
      pltpu.sync_copy(age_h.at[pl.ds(base, _EW)], i0_v.at[pl.ds(0, _EW)])
      pltpu.sync_copy(gen_h.at[pl.ds(base, _EW)], i1_v.at[pl.ds(0, _EW)])
      pltpu.sync_copy(occ_h.at[pl.ds(base, _EW)], i2_v.at[pl.ds(0, _EW)])
      pltpu.sync_copy(zip_h.at[pl.ds(base, _EW)], i3_v.at[pl.ds(0, _EW)])
      for g in range(_EBUF // 16):
        a16 = i0_v[pl.ds(g * 16, 16)]
        g16 = i1_v[pl.ds(g * 16, 16)]
        o16 = i2_v[pl.ds(g * 16, 16)]
        z16 = i3_v[pl.ds(g * 16, 16)]
        for e in range(16):
          u = g * 16 + e
          if u >= _EW:
            break
          ab = _bcast_lane(a16, e)
          gb = _bcast_lane(g16, e)
          ob = _bcast_lane(o16, e)
          zb = _bcast_lane(z16, e)
          addr1 = u_c1 + ab * u_ma + gb * u_mg + ob * u_mo + zb * u_mz1
          addr2 = u_c2 + zb * 18
          out_v[u, pl.ds(0, 16)] = plsc.load_gather(tab_v, [addr1])
          out_v[u, pl.ds(16, 16)] = plsc.load_gather(tab_v, [addr2])
      pltpu.sync_copy(out_v.at[pl.ds(0, _EW)], out_h.at[pl.ds(base, _EW)])

  @pl.when(c == 1)
  def _items():
    i_c1, i_mt, i_mte, i_c2 = _item_consts()
    pltpu.sync_copy(ti_h, tab_v.at[pl.ds(0, _TI_LEN)])

    @pl.loop(s, 625, step=NSUB)
    def _win(w):
      base = w * _EW
      pltpu.sync_copy(tit_h.at[pl.ds(base, _EW)], i0_v.at[pl.ds(0, _EW)])
      pltpu.sync_copy(tem_h.at[pl.ds(base, _EW)], i1_v.at[pl.ds(0, _EW)])
      pltpu.sync_copy(hum_h.at[pl.ds(base, _EW)], i2_v.at[pl.ds(0, _EW)])
      pltpu.sync_copy(win_h.at[pl.ds(base, _EW)], i3_v.at[pl.ds(0, _EW)])
      for g in range(_EBUF // 16):
        t16 = i0_v[pl.ds(g * 16, 16)]
        te16 = i1_v[pl.ds(g * 16, 16)]
        h16 = i2_v[pl.ds(g * 16, 16)]
        w16 = i3_v[pl.ds(g * 16, 16)]
        for e in range(16):
          u = g * 16 + e
          if u >= _EW:
            break
          tb = _bcast_lane(t16, e)
          teb = _bcast_lane(te16, e)
          hb = _bcast_lane(h16, e)
          wb = _bcast_lane(w16, e)
          addr1 = i_c1 + tb * i_mt + teb * i_mte
          addr2 = i_c2 + hb * i_mt + wb * i_mte
          out_v[u, pl.ds(0, 16)] = plsc.load_gather(tab_v, [addr1])
          out_v[u, pl.ds(16, 16)] = plsc.load_gather(tab_v, [addr2])
      pltpu.sync_copy(out_v.at[pl.ds(0, _EW)],
                      out_h.at[pl.ds(NU + base, _EW)])


def _embed_call(age, gen, occ, zp, tit, tem, hum, win, tu, ti):
  mesh = plsc.VectorSubcoreMesh(core_axis_name="c", subcore_axis_name="s")
  f = pl.kernel(
      _embed_body,
      out_type=jax.ShapeDtypeStruct((NN, 32), jnp.float32),
      mesh=mesh,
      compiler_params=pltpu.CompilerParams(needs_layout_passes=False, use_tc_tiling_on_sc=False),
      scratch_types=[
          pltpu.MemorySpace.VMEM((_TU_LEN,), jnp.float32),
          pltpu.MemorySpace.VMEM((_EBUF,), jnp.int32),
          pltpu.MemorySpace.VMEM((_EBUF,), jnp.int32),
          pltpu.MemorySpace.VMEM((_EBUF,), jnp.int32),
          pltpu.MemorySpace.VMEM((_EBUF,), jnp.int32),
          pltpu.MemorySpace.VMEM((_EBUF, 32), jnp.float32),
      ],
  )
  return f(age, gen, occ, zp, tit, tem, hum, win, tu, ti)


_W = 128
_SW = 16
_MW = _W * _SW
_NMW = -(-EH // _MW)
_EPS = _NMW * _MW


def _scale_sub(rows_v, b, v16s):
  for g in range(_W // 16):
    v16 = v16s[g]
    for e in range(16):
      r = g * 16 + e
      vb = _bcast_lane(v16, e)
      for j in range(4):
        rows_v[b, r, pl.ds(j * 16, 16)] = rows_v[b, r, pl.ds(j * 16, 16)] * vb


def _spmm_body(ecol_h, eval_h, erow_h, feat_h, out_h,
               col_v, val_v, row_v, rows_v, zz_v, g_sh, sem_a, sem_b):
  c = lax.axis_index("c")
  s = lax.axis_index("s")
  sems = (sem_a, sem_b)

  zf = (lax.iota(jnp.int32, 16) * 0).astype(jnp.float32)
  for i in range(25):
    for j in range(4):
      zz_v[i, pl.ds(j * 16, 16)] = zf

  @pl.loop(s, 1000, step=NSUB)
  def _zero(i):
    pltpu.sync_copy(zz_v, g_sh.at[pl.ds(i * 25, 25)])

  plsc.subcore_barrier()

  cnu = c * NU

  @pl.loop(s, _NMW, step=NSUB)
  def _mw(m):
    mrow = c * (_NMW * _SW) + m * _SW
    pltpu.sync_copy(ecol_h.at[pl.ds(mrow, _SW)], col_v)
    pltpu.sync_copy(eval_h.at[pl.ds(mrow, _SW)], val_v)
    pltpu.sync_copy(erow_h.at[pl.ds(mrow, _SW)], row_v)
    for i in range(_SW):
      for g in range(_W // 16):
        row_v[i, pl.ds(g * 16, 16)] = row_v[i, pl.ds(g * 16, 16)] - cnu
    pltpu.async_copy(feat_h.at[col_v.at[0]], rows_v.at[0], sem_a)
    pltpu.async_copy(feat_h.at[col_v.at[1]], rows_v.at[1], sem_b)

    @pl.loop(0, _SW, step=2)
    def _sub(j0):
      for b in range(2):
        j = j0 + b
        pltpu.make_async_copy(feat_h.at[col_v.at[j]], rows_v.at[b],
                              sems[b]).wait()
        v16s = [val_v[j, pl.ds(g * 16, 16)] for g in range(_W // 16)]
        _scale_sub(rows_v, b, v16s)
        pltpu.sync_copy(rows_v.at[b], g_sh.at[row_v.at[j]], add=True)

        @pl.when(j + 2 < _SW)
        def _pref():
          pltpu.async_copy(feat_h.at[col_v.at[j + 2]], rows_v.at[b], sems[b])

  plsc.subcore_barrier()

  @pl.loop(s, 625, step=NSUB)
  def _flush(i):
    pltpu.sync_copy(g_sh.at[pl.ds(i * 40, 40)],
                    out_h.at[pl.ds(cnu + i * 40, 40)])


def _spmm_call(ecol2, evl2, erow2, feat):
  mesh = plsc.VectorSubcoreMesh(core_axis_name="c", subcore_axis_name="s")
  f = pl.kernel(
      _spmm_body,
      out_type=jax.ShapeDtypeStruct((NN, DD), jnp.float32),
      mesh=mesh,
      compiler_params=pltpu.CompilerParams(needs_layout_passes=False, use_tc_tiling_on_sc=False),
      scratch_types=[
          pltpu.MemorySpace.VMEM((_SW, _W), jnp.int32),
          pltpu.MemorySpace.VMEM((_SW, _W), jnp.float32),
          pltpu.MemorySpace.VMEM((_SW, _W), jnp.int32),
          pltpu.MemorySpace.VMEM((2, _W, DD), jnp.float32),
          pltpu.MemorySpace.VMEM((25, DD), jnp.float32),
          pltpu.MemorySpace.VMEM_SHARED((NU, DD), jnp.float32),
          pltpu.SemaphoreType.DMA,
          pltpu.SemaphoreType.DMA,
      ],
  )
  return f(ecol2, evl2, erow2, feat)


_GW = BB // (NCORE * NSUB)


def _gather_body(users_h, items_h, f0_h, f1_h, f2_h,
                 ou0_h, ou1_h, ou2_h, oi0_h, oi1_h, oi2_h,
                 idx_v, rows_v, sem):
  c = lax.axis_index("c")
  s = lax.axis_index("s")
  w = s * NCORE + c
  base = w * _GW

  pltpu.sync_copy(users_h.at[pl.ds(base, _GW)], idx_v)
  for (f_h, o_h) in ((f0_h, ou0_h), (f1_h, ou1_h), (f2_h, ou2_h)):
    pltpu.async_copy(f_h.at[idx_v], rows_v, sem).wait()
    pltpu.sync_copy(rows_v, o_h.at[pl.ds(base, _GW)])

  pltpu.sync_copy(items_h.at[pl.ds(base, _GW)], idx_v)
  for g in range(_GW // 16):
    idx_v[pl.ds(g * 16, 16)] = idx_v[pl.ds(g * 16, 16)] + NU
  for (f_h, o_h) in ((f0_h, oi0_h), (f1_h, oi1_h), (f2_h, oi2_h)):
    pltpu.async_copy(f_h.at[idx_v], rows_v, sem).wait()
    pltpu.sync_copy(rows_v, o_h.at[pl.ds(base, _GW)])


def _gather_call(users, items, f0, f1, f2):
  mesh = plsc.VectorSubcoreMesh(core_axis_name="c", subcore_axis_name="s")
  ot = jax.ShapeDtypeStruct((BB, DD), jnp.float32)
  f = pl.kernel(
      _gather_body,
      out_type=(ot, ot, ot, ot, ot, ot),
      mesh=mesh,
      compiler_params=pltpu.CompilerParams(needs_layout_passes=False, use_tc_tiling_on_sc=False),
      scratch_types=[
          pltpu.MemorySpace.VMEM((_GW,), jnp.int32),
          pltpu.MemorySpace.VMEM((_GW, DD), jnp.float32),
          pltpu.SemaphoreType.DMA,
      ],
  )
  return f(users, items, f0, f1, f2)


_RB = 1000


def _dense_body(g_ref, f_ref, w1_ref, w2_ref, b1_ref, b2_ref, o_ref):
  g = g_ref[...]
  f = f_ref[...]
  dn = (((1,), (1,)), ((), ()))
  acc = lax.dot_general(g + f, w1_ref[...], dn,
                        preferred_element_type=jnp.float32)
  acc = acc + lax.dot_general(g * f, w2_ref[...], dn,
                              preferred_element_type=jnp.float32)
  acc = acc + b1_ref[...] + b2_ref[...]
  o_ref[...] = jnp.where(acc > 0, acc, 0.01 * acc)


def _dense_call(g, f, w1, w2, b1, b2):
  grid = (NN // _RB,)
  return pl.pallas_call(
      _dense_body,
      grid=grid,
      in_specs=[
          pl.BlockSpec((_RB, DD), lambda i: (i, 0)),
          pl.BlockSpec((_RB, DD), lambda i: (i, 0)),
          pl.BlockSpec((DD, DD), lambda i: (0, 0)),
          pl.BlockSpec((DD, DD), lambda i: (0, 0)),
          pl.BlockSpec((1, DD), lambda i: (0, 0)),
          pl.BlockSpec((1, DD), lambda i: (0, 0)),
      ],
      out_specs=pl.BlockSpec((_RB, DD), lambda i: (i, 0)),
      out_shape=jax.ShapeDtypeStruct((NN, DD), jnp.float32),
  )(g, f, w1, w2, b1.reshape(1, DD), b2.reshape(1, DD))


def _final_body(u0_ref, u1_ref, u2_ref, i0_ref, i1_ref, i2_ref,
                wt_ref, bt_ref, o_ref):
  wt = wt_ref[...]
  dn = (((1,), (1,)), ((), ()))
  ue = bt_ref[...]
  ie = bt_ref[...]
  for l, (u_ref, i_ref) in enumerate(((u0_ref, i0_ref), (u1_ref, i1_ref),
                                      (u2_ref, i2_ref))):
    wtl = wt[:, l * DD:(l + 1) * DD]
    ue = ue + lax.dot_general(u_ref[...], wtl, dn,
                              preferred_element_type=jnp.float32)
    ie = ie + lax.dot_general(i_ref[...], wtl, dn,
                              preferred_element_type=jnp.float32)
  o_ref[...] = jnp.sum(ue * ie, axis=1)


def _final_call(u0, u1, u2, i0, i1, i2, wt, bt):
  return pl.pallas_call(
      _final_body,
      grid=(1,),
      in_specs=[pl.BlockSpec((BB, DD), lambda i: (0, 0))] * 6
      + [pl.BlockSpec((DD, 3 * DD), lambda i: (0, 0)),
         pl.BlockSpec((1, DD), lambda i: (0, 0))],
      out_specs=pl.BlockSpec((BB,), lambda i: (0,)),
      out_shape=jax.ShapeDtypeStruct((BB,), jnp.float32),
  )(u0, u1, u2, i0, i1, i2, wt, bt.reshape(1, DD))


def kernel(users, items, edge_row, edge_col, edge_val, user_age, user_gender,
           user_occ, user_zip, item_title, item_temp, item_hum, item_wind,
           user_id_tab, user_age_tab, user_gender_tab, user_occ_tab,
           user_zip_tab, item_id_tab, item_title_tab, item_temp_tab,
           item_hum_tab, item_wind_tab, W1, b1, W2, b2, Wt, bt):
  tu = jnp.concatenate([
      user_age_tab.reshape(-1), user_gender_tab.reshape(-1),
      user_occ_tab.reshape(-1), user_zip_tab.reshape(-1),
      jnp.zeros((_TU_LEN - 18684,), jnp.float32)])
  ti = jnp.concatenate([
      item_title_tab.reshape(-1), item_temp_tab.reshape(-1),
      item_hum_tab.reshape(-1), item_wind_tab.reshape(-1)])

  emb32 = _embed_call(
      user_age.astype(jnp.int32), user_gender.astype(jnp.int32),
      user_occ.astype(jnp.int32), user_zip.astype(jnp.int32),
      item_title.astype(jnp.int32), item_temp.astype(jnp.int32),
      item_hum.astype(jnp.int32), item_wind.astype(jnp.int32), tu, ti)

  ids = jnp.concatenate([user_id_tab, item_id_tab], axis=0)
  feat = jnp.concatenate([ids, emb32], axis=1)

  ecol = edge_col.astype(jnp.int32)
  erow = edge_row.astype(jnp.int32)
  npad = _EPS - EH
  ar = jnp.arange(npad, dtype=jnp.int32)
  padc = ar % NN
  padr0 = ar % NU
  padr1 = NU + padr0
  padv = jnp.zeros((npad,), jnp.float32)
  ecol2 = jnp.concatenate([ecol[:EH], padc, ecol[EH:], padc]).reshape(-1, _W)
  erow2 = jnp.concatenate([erow[:EH], padr0, erow[EH:], padr1]).reshape(-1, _W)
  evl2 = jnp.concatenate([edge_val[:EH], padv,
                          edge_val[EH:], padv]).reshape(-1, _W)

  feats = [feat]
  for l in range(2):
    g = _spmm_call(ecol2, evl2, erow2, feat)
    feat = _dense_call(g, feat, W1[l], W2[l], b1[l], b2[l])
    feats.append(feat)

  gu0, gu1, gu2, gi0, gi1, gi2 = _gather_call(
      users.astype(jnp.int32), items.astype(jnp.int32),
      feats[0], feats[1], feats[2])

  return _final_call(gu0, gu1, gu2, gi0, gi1, gi2, Wt, bt)


# speedup vs baseline: 9.5554x; 1.0678x over previous
import functools

import jax
import jax.numpy as jnp
import numpy as np
from jax import lax
from jax.experimental import pallas as pl
from jax.experimental.pallas import tpu as pltpu
from jax.experimental.pallas import tpu_sc as plsc

NU = 25000
NI = 25000
NN = NU + NI
EH = 400000
DD = 64
BB = 4096

NCORE = 2
NSUB = 16
LANE = 16

def _bcast_lane(v16, e):
  return lax.broadcast_in_dim(v16[e], (16,), ())


_U_AB, _U_GB, _U_OB, _U_ZB = 0, 480, 484, 684
_I_TB, _I_TEB, _I_HB, _I_WB = 0, 16000, 16400, 17200
_TU_LEN = 18688
_TI_LEN = 17520

_EMW = 640
_ENP = 25600


def _user_consts():
  i16 = lax.iota(jnp.int32, 16)
  in_a = i16 < 4
  in_g = jnp.logical_and(i16 >= 4, i16 < 6)
  in_o = jnp.logical_and(i16 >= 6, i16 < 14)
  c1 = jnp.where(in_a, i16,
                 jnp.where(in_g, i16 + (_U_GB - 4),
                           jnp.where(in_o, i16 + (_U_OB - 6),
                                     i16 + (_U_ZB - 14))))
  m_a = jnp.where(in_a, 4, 0)
  m_g = jnp.where(in_g, 2, 0)
  m_o = jnp.where(in_o, 8, 0)
  m_z1 = jnp.where(i16 >= 14, 18, 0)
  c2 = i16 + (_U_ZB + 2)
  return c1, m_a, m_g, m_o, m_z1, c2


def _item_consts():
  i16 = lax.iota(jnp.int32, 16)
  lo = i16 < 8
  c1 = jnp.where(lo, i16, i16 + (_I_TEB - 8))
  m_t = jnp.where(lo, 8, 0)
  m_te = 8 - m_t
  c2 = jnp.where(lo, i16 + _I_HB, i16 + (_I_WB - 8))
  return c1, m_t, m_te, c2


def _embed_body(age_h, gen_h, occ_h, zip_h, tit_h, tem_h, hum_h, win_h,
                tu_h, ti_h, out_h,
                tab_v, i0_v, i1_v, i2_v, i3_v, out_v):
  c = lax.axis_index("c")
  s = lax.axis_index("s")

  @pl.when(c == 0)
  def _users():
    u_c1, u_ma, u_mg, u_mo, u_mz1, u_c2 = _user_consts()
    pltpu.sync_copy(tu_h, tab_v.at[pl.ds(0, _TU_LEN)])

    @pl.loop(s, _ENP // _EMW, step=NSUB)
    def _mw(m):
      base = m * _EMW
      pltpu.sync_copy(age_h.at[pl.ds(base, _EMW)], i0_v)
      pltpu.sync_copy(gen_h.at[pl.ds(base, _EMW)], i1_v)
      pltpu.sync_copy(occ_h.at[pl.ds(base, _EMW)], i2_v)
      pltpu.sync_copy(zip_h.at[pl.ds(base, _EMW)], i3_v)

      @pl.loop(0, _EMW // 16)
      def _grp(gg):
        a16 = i0_v[pl.ds(gg * 16, 16)]
        g16 = i1_v[pl.ds(gg * 16, 16)]
        o16 = i2_v[pl.ds(gg * 16, 16)]
        z16 = i3_v[pl.ds(gg * 16, 16)]
        for e in range(16):
          u = gg * 16 + e
          ab = _bcast_lane(a16, e)
          gb = _bcast_lane(g16, e)
          ob = _bcast_lane(o16, e)
          zb = _bcast_lane(z16, e)
          addr1 = u_c1 + ab * u_ma + gb * u_mg + ob * u_mo + zb * u_mz1
          addr2 = u_c2 + zb * 18
          out_v[u, pl.ds(0, 16)] = plsc.load_gather(tab_v, [addr1])
          out_v[u, pl.ds(16, 16)] = plsc.load_gather(tab_v, [addr2])

      pltpu.sync_copy(out_v, out_h.at[0, pl.ds(base, _EMW)])

  @pl.when(c == 1)
  def _items():
    i_c1, i_mt, i_mte, i_c2 = _item_consts()
    pltpu.sync_copy(ti_h, tab_v.at[pl.ds(0, _TI_LEN)])

    @pl.loop(s, _ENP // _EMW, step=NSUB)
    def _mw(m):
      base = m * _EMW
      pltpu.sync_copy(tit_h.at[pl.ds(base, _EMW)], i0_v)
      pltpu.sync_copy(tem_h.at[pl.ds(base, _EMW)], i1_v)
      pltpu.sync_copy(hum_h.at[pl.ds(base, _EMW)], i2_v)
      pltpu.sync_copy(win_h.at[pl.ds(base, _EMW)], i3_v)

      @pl.loop(0, _EMW // 16)
      def _grp(gg):
        t16 = i0_v[pl.ds(gg * 16, 16)]
        te16 = i1_v[pl.ds(gg * 16, 16)]
        h16 = i2_v[pl.ds(gg * 16, 16)]
        w16 = i3_v[pl.ds(gg * 16, 16)]
        for e in range(16):
          u = gg * 16 + e
          tb = _bcast_lane(t16, e)
          teb = _bcast_lane(te16, e)
          hb = _bcast_lane(h16, e)
          wb = _bcast_lane(w16, e)
          addr1 = i_c1 + tb * i_mt + teb * i_mte
          addr2 = i_c2 + hb * i_mt + wb * i_mte
          out_v[u, pl.ds(0, 16)] = plsc.load_gather(tab_v, [addr1])
          out_v[u, pl.ds(16, 16)] = plsc.load_gather(tab_v, [addr2])

      pltpu.sync_copy(out_v, out_h.at[1, pl.ds(base, _EMW)])


def _embed_call(age, gen, occ, zp, tit, tem, hum, win, tu, ti):
  mesh = plsc.VectorSubcoreMesh(core_axis_name="c", subcore_axis_name="s")
  f = pl.kernel(
      _embed_body,
      out_type=jax.ShapeDtypeStruct((2, _ENP, 32), jnp.float32),
      mesh=mesh,
      compiler_params=pltpu.CompilerParams(needs_layout_passes=False, use_tc_tiling_on_sc=False),
      scratch_types=[
          pltpu.MemorySpace.VMEM((_TU_LEN,), jnp.float32),
          pltpu.MemorySpace.VMEM((_EMW,), jnp.int32),
          pltpu.MemorySpace.VMEM((_EMW,), jnp.int32),
          pltpu.MemorySpace.VMEM((_EMW,), jnp.int32),
          pltpu.MemorySpace.VMEM((_EMW,), jnp.int32),
          pltpu.MemorySpace.VMEM((_EMW, 32), jnp.float32),
      ],
  )
  return f(age, gen, occ, zp, tit, tem, hum, win, tu, ti)


_W = 128
_SW = 16
_MW = _W * _SW
_NMW = -(-EH // _MW)
_EPS = _NMW * _MW

def _spmm_body(ecol_h, eval_h, erow_h, feat_h, out_h,
               col_v, val_v, row_v, rows_v, g_sh, sem_a, sem_b):
  c = lax.axis_index("c")
  s = lax.axis_index("s")
  sems = (sem_a, sem_b)

  zf = (lax.iota(jnp.int32, 16) * 0).astype(jnp.float32)
  for i in range(125):
    for j in range(4):
      rows_v[0, i, pl.ds(j * 16, 16)] = zf

  @pl.loop(s, 200, step=NSUB)
  def _zero(i):
    pltpu.sync_copy(rows_v.at[0, pl.ds(0, 125)], g_sh.at[pl.ds(i * 125, 125)])

  plsc.subcore_barrier()

  cnu = c * NU

  @pl.loop(s, _NMW, step=NSUB)
  def _mw(m):
    mrow = c * (_NMW * _SW) + m * _SW
    pltpu.sync_copy(ecol_h.at[pl.ds(mrow, _SW)], col_v)
    pltpu.sync_copy(eval_h.at[pl.ds(mrow, _SW)], val_v)
    pltpu.sync_copy(erow_h.at[pl.ds(mrow, _SW)], row_v)
    for i in range(_SW):
      for g in range(_W // 16):
        row_v[i, pl.ds(g * 16, 16)] = row_v[i, pl.ds(g * 16, 16)] - cnu
    pltpu.async_copy(feat_h.at[col_v.at[0]], rows_v.at[0], sem_a)
    pltpu.async_copy(feat_h.at[col_v.at[1]], rows_v.at[1], sem_b)

    @pl.loop(0, _SW, step=2)
    def _sub(j0):
      for b in range(2):
        j = j0 + b
        pltpu.make_async_copy(feat_h.at[col_v.at[j]], rows_v.at[b],
                              sems[b]).wait()
        for g in range(_W // 16):
          v16 = val_v[j, pl.ds(g * 16, 16)]
          for e in range(16):
            r = g * 16 + e
            vb = _bcast_lane(v16, e)
            for q in range(DD // 16):
              rows_v[b, r, pl.ds(q * 16, 16)] = (
                  rows_v[b, r, pl.ds(q * 16, 16)] * vb)
        pltpu.sync_copy(rows_v.at[b], g_sh.at[row_v.at[j]], add=True)

        @pl.when(j + 2 < _SW)
        def _pref():
          pltpu.async_copy(feat_h.at[col_v.at[j + 2]], rows_v.at[b], sems[b])

  plsc.subcore_barrier()

  @pl.loop(s, 200, step=NSUB)
  def _flush(i):
    pltpu.sync_copy(g_sh.at[pl.ds(i * 125, 125)],
                    out_h.at[pl.ds(cnu + i * 125, 125)])


def _spmm_call(ecol2, evl2, erow2, feat):
  mesh = plsc.VectorSubcoreMesh(core_axis_name="c", subcore_axis_name="s")
  f = pl.kernel(
      _spmm_body,
      out_type=jax.ShapeDtypeStruct((NN, DD), jnp.float32),
      mesh=mesh,
      compiler_params=pltpu.CompilerParams(needs_layout_passes=False, use_tc_tiling_on_sc=False),
      scratch_types=[
          pltpu.MemorySpace.VMEM((_SW, _W), jnp.int32),
          pltpu.MemorySpace.VMEM((_SW, _W), jnp.float32),
          pltpu.MemorySpace.VMEM((_SW, _W), jnp.int32),
          pltpu.MemorySpace.VMEM((2, _W, DD), jnp.float32),
          pltpu.MemorySpace.VMEM_SHARED((NU, DD), jnp.float32),
          pltpu.SemaphoreType.DMA,
          pltpu.SemaphoreType.DMA,
      ],
  )
  return f(ecol2, evl2, erow2, feat)


_GW = BB // (NCORE * NSUB)


def _gather_body(users_h, items_h, f0_h, f1_h, f2_h,
                 ou0_h, ou1_h, ou2_h, oi0_h, oi1_h, oi2_h,
                 idx_v, rows_v, sem):
  c = lax.axis_index("c")
  s = lax.axis_index("s")
  w = s * NCORE + c
  base = w * _GW

  pltpu.sync_copy(users_h.at[pl.ds(base, _GW)], idx_v)
  for (f_h, o_h) in ((f0_h, ou0_h), (f1_h, ou1_h), (f2_h, ou2_h)):
    pltpu.async_copy(f_h.at[idx_v], rows_v, sem).wait()
    pltpu.sync_copy(rows_v, o_h.at[pl.ds(base, _GW)])

  pltpu.sync_copy(items_h.at[pl.ds(base, _GW)], idx_v)
  for g in range(_GW // 16):
    idx_v[pl.ds(g * 16, 16)] = idx_v[pl.ds(g * 16, 16)] + NU
  for (f_h, o_h) in ((f0_h, oi0_h), (f1_h, oi1_h), (f2_h, oi2_h)):
    pltpu.async_copy(f_h.at[idx_v], rows_v, sem).wait()
    pltpu.sync_copy(rows_v, o_h.at[pl.ds(base, _GW)])


def _gather_call(users, items, f0, f1, f2):
  mesh = plsc.VectorSubcoreMesh(core_axis_name="c", subcore_axis_name="s")
  ot = jax.ShapeDtypeStruct((BB, DD), jnp.float32)
  f = pl.kernel(
      _gather_body,
      out_type=(ot, ot, ot, ot, ot, ot),
      mesh=mesh,
      compiler_params=pltpu.CompilerParams(needs_layout_passes=False, use_tc_tiling_on_sc=False),
      scratch_types=[
          pltpu.MemorySpace.VMEM((_GW,), jnp.int32),
          pltpu.MemorySpace.VMEM((_GW, DD), jnp.float32),
          pltpu.SemaphoreType.DMA,
      ],
  )
  return f(users, items, f0, f1, f2)


_RB = 1000


def _dense_body(g_ref, f_ref, w1_ref, w2_ref, b1_ref, b2_ref, o_ref):
  g = g_ref[...]
  f = f_ref[...]
  dn = (((1,), (1,)), ((), ()))
  acc = lax.dot_general(g + f, w1_ref[...], dn,
                        preferred_element_type=jnp.float32)
  acc = acc + lax.dot_general(g * f, w2_ref[...], dn,
                              preferred_element_type=jnp.float32)
  acc = acc + b1_ref[...] + b2_ref[...]
  o_ref[...] = jnp.where(acc > 0, acc, 0.01 * acc)


def _dense_call(g, f, w1, w2, b1, b2):
  grid = (NN // _RB,)
  return pl.pallas_call(
      _dense_body,
      grid=grid,
      in_specs=[
          pl.BlockSpec((_RB, DD), lambda i: (i, 0)),
          pl.BlockSpec((_RB, DD), lambda i: (i, 0)),
          pl.BlockSpec((DD, DD), lambda i: (0, 0)),
          pl.BlockSpec((DD, DD), lambda i: (0, 0)),
          pl.BlockSpec((1, DD), lambda i: (0, 0)),
          pl.BlockSpec((1, DD), lambda i: (0, 0)),
      ],
      out_specs=pl.BlockSpec((_RB, DD), lambda i: (i, 0)),
      out_shape=jax.ShapeDtypeStruct((NN, DD), jnp.float32),
  )(g, f, w1, w2, b1.reshape(1, DD), b2.reshape(1, DD))


def _final_body(u0_ref, u1_ref, u2_ref, i0_ref, i1_ref, i2_ref,
                wt_ref, bt_ref, o_ref):
  wt = wt_ref[...]
  dn = (((1,), (1,)), ((), ()))
  ue = bt_ref[...]
  ie = bt_ref[...]
  for l, (u_ref, i_ref) in enumerate(((u0_ref, i0_ref), (u1_ref, i1_ref),
                                      (u2_ref, i2_ref))):
    wtl = wt[:, l * DD:(l + 1) * DD]
    ue = ue + lax.dot_general(u_ref[...], wtl, dn,
                              preferred_element_type=jnp.float32)
    ie = ie + lax.dot_general(i_ref[...], wtl, dn,
                              preferred_element_type=jnp.float32)
  o_ref[...] = jnp.sum(ue * ie, axis=1)


def _final_call(u0, u1, u2, i0, i1, i2, wt, bt):
  return pl.pallas_call(
      _final_body,
      grid=(1,),
      in_specs=[pl.BlockSpec((BB, DD), lambda i: (0, 0))] * 6
      + [pl.BlockSpec((DD, 3 * DD), lambda i: (0, 0)),
         pl.BlockSpec((1, DD), lambda i: (0, 0))],
      out_specs=pl.BlockSpec((BB,), lambda i: (0,)),
      out_shape=jax.ShapeDtypeStruct((BB,), jnp.float32),
  )(u0, u1, u2, i0, i1, i2, wt, bt.reshape(1, DD))


def kernel(users, items, edge_row, edge_col, edge_val, user_age, user_gender,
           user_occ, user_zip, item_title, item_temp, item_hum, item_wind,
           user_id_tab, user_age_tab, user_gender_tab, user_occ_tab,
           user_zip_tab, item_id_tab, item_title_tab, item_temp_tab,
           item_hum_tab, item_wind_tab, W1, b1, W2, b2, Wt, bt):
  tu = jnp.concatenate([
      user_age_tab.reshape(-1), user_gender_tab.reshape(-1),
      user_occ_tab.reshape(-1), user_zip_tab.reshape(-1),
      jnp.zeros((_TU_LEN - 18684,), jnp.float32)])
  ti = jnp.concatenate([
      item_title_tab.reshape(-1), item_temp_tab.reshape(-1),
      item_hum_tab.reshape(-1), item_wind_tab.reshape(-1)])

  zpad = jnp.zeros((_ENP - NU,), jnp.int32)
  def _p(x):
    return jnp.concatenate([x.astype(jnp.int32), zpad])

  emb2 = _embed_call(
      _p(user_age), _p(user_gender), _p(user_occ), _p(user_zip),
      _p(item_title), _p(item_temp), _p(item_hum), _p(item_wind), tu, ti)
  emb32 = jnp.concatenate([emb2[0, :NU], emb2[1, :NU]], axis=0)

  ids = jnp.concatenate([user_id_tab, item_id_tab], axis=0)
  feat = jnp.concatenate([ids, emb32], axis=1)

  ecol = edge_col.astype(jnp.int32)
  erow = edge_row.astype(jnp.int32)
  npad = _EPS - EH
  ar = jnp.arange(npad, dtype=jnp.int32)
  padc = ar % NN
  padr0 = ar % NU
  padr1 = NU + padr0
  padv = jnp.zeros((npad,), jnp.float32)
  ecol2 = jnp.concatenate([ecol[:EH], padc, ecol[EH:], padc]).reshape(-1, _W)
  erow2 = jnp.concatenate([erow[:EH], padr0, erow[EH:], padr1]).reshape(-1, _W)
  evl2 = jnp.concatenate([edge_val[:EH], padv,
                          edge_val[EH:], padv]).reshape(-1, _W)

  feats = [feat]
  for l in range(2):
    g = _spmm_call(ecol2, evl2, erow2, feat)
    feat = _dense_call(g, feat, W1[l], W2[l], b1[l], b2[l])
    feats.append(feat)

  gu0, gu1, gu2, gi0, gi1, gi2 = _gather_call(
      users.astype(jnp.int32), items.astype(jnp.int32),
      feats[0], feats[1], feats[2])

  return _final_call(gu0, gu1, gu2, gi0, gi1, gi2, Wt, bt)
